# Initial kernel scaffold; baseline (speedup 1.0000x reference)
import jax
import jax.numpy as jnp
from jax.experimental import pallas as pl


def kernel(x, scale, gate_w, gate_b, mlp1_w, mlp1_b, mlp2_w, mlp2_b):
    raise NotImplementedError("write your pallas kernel here")


# speedup vs baseline: 9.8181x; 9.8181x over previous
import functools

import jax
import jax.numpy as jnp
from jax.experimental import pallas as pl
from jax.experimental.pallas import tpu as pltpu

LIMIT = 7.0
ALPHA = 1.702
EPS = 1e-5
TM = 16
TP = 2048


def _moe_body(cnt_ref, st_ref,
              t_ref, x_ref, tok_ref, wp_ref, w1_ref, b1g_ref, b1l_ref,
              w2_ref, b2_ref,
              out_ref, xs_ref, y_ref, gw_ref, pg_ref, pl_ref):
    e = pl.program_id(0)
    n_experts = pl.num_programs(0)
    f_dim = pg_ref.shape[1]
    tp = xs_ref.shape[0]
    t_tok = t_ref.shape[0]

    @pl.when(e == 0)
    def _init():
        rows = jax.lax.broadcasted_iota(jnp.int32, (2 * f_dim, f_dim), 0)
        cols = jax.lax.broadcasted_iota(jnp.int32, (2 * f_dim, f_dim), 1)
        pg_ref[...] = (rows == 2 * cols).astype(jnp.bfloat16)
        pl_ref[...] = (rows == 2 * cols + 1).astype(jnp.bfloat16)
        col = jax.lax.broadcasted_iota(jnp.int32, (tp, t_tok), 1)
        onehot = (col == tok_ref[...]).astype(jnp.bfloat16)
        gw_ref[...] = onehot * wp_ref[...]
        y_ref[...] = jnp.zeros_like(y_ref)
        xs_ref[...] = jax.lax.dot_general(
            onehot, t_ref[...], (((1,), (0,)), ((), ())),
            preferred_element_type=jnp.float32).astype(jnp.bfloat16)

    n_p = cnt_ref[e]
    s_p = st_ref[e]
    w1 = w1_ref[0]
    w2 = w2_ref[0]

    def tile_body(i, carry):
        base = pl.multiple_of(s_p + i * TM, TM)
        xs = xs_ref[pl.ds(base, TM), :]
        h = jax.lax.dot_general(xs, w1, (((1,), (1,)), ((), ())),
                                preferred_element_type=jnp.float32)
        h = h.astype(jnp.bfloat16)
        hg = jax.lax.dot_general(h, pg_ref[...], (((1,), (0,)), ((), ())),
                                 preferred_element_type=jnp.float32)
        hl = jax.lax.dot_general(h, pl_ref[...], (((1,), (0,)), ((), ())),
                                 preferred_element_type=jnp.float32)
        hg = (hg.astype(jnp.bfloat16) + b1g_ref[0]).astype(jnp.float32)
        hl = (hl.astype(jnp.bfloat16) + b1l_ref[0]).astype(jnp.float32)
        hg = jnp.minimum(hg, LIMIT)
        hl = jnp.clip(hl, -LIMIT, LIMIT)
        act = (hg * jax.nn.sigmoid(ALPHA * hg) * (hl + 1.0)).astype(jnp.bfloat16)
        y = jax.lax.dot_general(act, w2, (((1,), (1,)), ((), ())),
                                preferred_element_type=jnp.float32)
        y_ref[pl.ds(base, TM), :] = y.astype(jnp.bfloat16) + b2_ref[0]
        return carry

    jax.lax.fori_loop(0, n_p // TM, tile_body, 0)

    @pl.when(e == n_experts - 1)
    def _fin():
        delta = jax.lax.dot_general(
            gw_ref[...], y_ref[...], (((0,), (0,)), ((), ())),
            preferred_element_type=jnp.float32)
        out_ref[...] = (x_ref[...].astype(jnp.float32) + delta
                        ).astype(jnp.bfloat16)


def kernel(x, scale, gate_w, gate_b, mlp1_w, mlp1_b, mlp2_w, mlp2_b):
    bsz, seq, h = x.shape
    t_tok = bsz * seq
    e_num = gate_w.shape[0]
    f_dim = mlp2_w.shape[2]
    x2 = x.reshape(t_tok, h)

    t32 = x2.astype(jnp.float32)
    t32 = t32 * jax.lax.rsqrt(jnp.mean(t32 * t32, axis=-1, keepdims=True)
                              + EPS)
    t_norm = (t32 * scale).astype(x.dtype)
    g = t_norm @ gate_w.T + gate_b

    vals, idx = jax.lax.top_k(g, 2)
    wts = jax.nn.softmax(vals, axis=-1)
    ea = idx.reshape(-1)
    counts = jnp.zeros((e_num,), jnp.int32).at[ea].add(1)
    counts_p = ((counts + TM - 1) // TM) * TM
    zero = jnp.zeros((1,), jnp.int32)
    starts = jnp.concatenate([zero, jnp.cumsum(counts)[:-1]])
    starts_p = jnp.concatenate([zero, jnp.cumsum(counts_p)[:-1]])
    order = jnp.argsort(ea, stable=True)
    e_sorted = ea[order]
    rank = jnp.arange(ea.shape[0], dtype=jnp.int32) - starts[e_sorted]
    pos = starts_p[e_sorted] + rank
    tok_pad = jnp.zeros((TP, 1), jnp.int32).at[pos, 0].set(
        (order // 2).astype(jnp.int32))
    w_pad = jnp.zeros((TP, 1), jnp.bfloat16).at[pos, 0].set(
        wts.reshape(-1)[order])

    b1r = mlp1_b.reshape(e_num, f_dim, 2)
    b1g = b1r[:, :, 0].reshape(e_num, 1, f_dim)
    b1l = b1r[:, :, 1].reshape(e_num, 1, f_dim)
    b2r = mlp2_b.reshape(e_num, 1, h)

    smem = functools.partial(pl.BlockSpec, memory_space=pltpu.SMEM)
    out = pl.pallas_call(
        _moe_body,
        grid=(e_num,),
        in_specs=[
            smem(), smem(),
            pl.BlockSpec((t_tok, h), lambda e: (0, 0)),
            pl.BlockSpec((t_tok, h), lambda e: (0, 0)),
            pl.BlockSpec((TP, 1), lambda e: (0, 0)),
            pl.BlockSpec((TP, 1), lambda e: (0, 0)),
            pl.BlockSpec((1, 2 * f_dim, h), lambda e: (e, 0, 0)),
            pl.BlockSpec((1, 1, f_dim), lambda e: (e, 0, 0)),
            pl.BlockSpec((1, 1, f_dim), lambda e: (e, 0, 0)),
            pl.BlockSpec((1, h, f_dim), lambda e: (e, 0, 0)),
            pl.BlockSpec((1, 1, h), lambda e: (e, 0, 0)),
        ],
        out_specs=pl.BlockSpec((t_tok, h), lambda e: (0, 0)),
        out_shape=jax.ShapeDtypeStruct((t_tok, h), jnp.bfloat16),
        scratch_shapes=[pltpu.VMEM((TP, h), jnp.bfloat16),
                        pltpu.VMEM((TP, h), jnp.bfloat16),
                        pltpu.VMEM((TP, t_tok), jnp.bfloat16),
                        pltpu.VMEM((2 * f_dim, f_dim), jnp.bfloat16),
                        pltpu.VMEM((2 * f_dim, f_dim), jnp.bfloat16)],
    )(counts_p, starts_p, t_norm, x2, tok_pad, w_pad,
      mlp1_w, b1g, b1l, mlp2_w, b2r)
    return out.reshape(bsz, seq, h)
